# Initial kernel scaffold; baseline (speedup 1.0000x reference)
import jax
import jax.numpy as jnp
from jax.experimental import pallas as pl


def kernel(pred, target):
    raise NotImplementedError("write your pallas kernel here")


# speedup vs baseline: 2.4012x; 2.4012x over previous
import functools
import math

import jax
import jax.numpy as jnp
from jax.experimental import pallas as pl

_SMOOTH = 0.1
_PAD = 0


def _ls_kernel(tgt_ref, pred_ref, out_ref, *, eps, c0):
    i = pl.program_id(0)
    pred = pred_ref[...]
    tgt = tgt_ref[...]
    cols = jax.lax.broadcasted_iota(jnp.int32, pred.shape, 1)
    pt = jnp.sum(jnp.where(cols == tgt, pred, 0.0), axis=1, keepdims=True)
    p0 = jnp.sum(jnp.where(cols == _PAD, pred, 0.0), axis=1, keepdims=True)
    rowsum = jnp.sum(pred, axis=1, keepdims=True)
    contrib = jnp.where(tgt != _PAD,
                        c0 - eps * rowsum + eps * p0 + (eps - (1.0 - _SMOOTH)) * pt,
                        0.0)
    s = jnp.sum(contrib, axis=0, keepdims=True)

    @pl.when(i == 0)
    def _():
        out_ref[...] = s

    @pl.when(i != 0)
    def _():
        out_ref[...] += s


def kernel(pred, target):
    B, S, V = pred.shape
    R = B * S
    pred2 = pred.reshape(R, V)
    tgt2 = target.reshape(R, 1)
    RB = 256
    G = R // RB
    eps = _SMOOTH / (V - 2)
    c0 = _SMOOTH * math.log(eps) + (1.0 - _SMOOTH) * math.log(1.0 - _SMOOTH)
    out = pl.pallas_call(
        functools.partial(_ls_kernel, eps=eps, c0=c0),
        grid=(G,),
        in_specs=[
            pl.BlockSpec((RB, 1), lambda i: (i, 0)),
            pl.BlockSpec((RB, V), lambda i: (i, 0)),
        ],
        out_specs=pl.BlockSpec((1, 1), lambda i: (0, 0)),
        out_shape=jax.ShapeDtypeStruct((1, 1), jnp.float32),
    )(tgt2, pred2)
    return out[0, 0]
